# Initial kernel scaffold; baseline (speedup 1.0000x reference)
import jax
import jax.numpy as jnp
from jax.experimental import pallas as pl


def kernel(x, uu_sim, ii_sim, hist, user_embedding, item_embedding, user_cnn_w, user_cnn_b, item_cnn_w, item_cnn_b, int_W, int_b, fg_W, fg_b, W_zh, W_zx, b_z, W_rh, W_rx, b_r, W_hh, W_hx, b_h, W_ch, b_c, fc_W, fc_b, pred_W, pred_b):
    raise NotImplementedError("write your pallas kernel here")


# speedup vs baseline: 6.1392x; 6.1392x over previous
import functools

import jax
import jax.numpy as jnp
from jax.experimental import pallas as pl

NUM_TIMES = 64
EMBED_DIM = 64
TOP_K = 10
TIME_WINDOW = 8

_ROW_BLK = 256
_BATCH_BLK = 512


def _topk_nei_kernel(sim_ref, emb_ref, w_ref, out_ref):
    vals = sim_ref[...]
    r, n = vals.shape
    col = jax.lax.broadcasted_iota(jnp.int32, (r, n), 1)
    acc = jnp.zeros((r, n), jnp.float32)
    for k in range(TOP_K):
        m = jnp.max(vals, axis=1, keepdims=True)
        idx = jnp.min(jnp.where(vals == m, col, n), axis=1, keepdims=True)
        hit = col == idx
        acc = acc + jnp.where(hit, w_ref[0, k], 0.0)
        vals = jnp.where(hit, -jnp.inf, vals)
    nei = jnp.dot(acc, emb_ref[...], preferred_element_type=jnp.float32,
                precision=jax.lax.Precision.HIGHEST)
    out_ref[...] = nei + w_ref[0, TOP_K]


def _neighbor_combine(sim, emb, cnn_w, cnn_b):
    n = sim.shape[0]
    wpack = jnp.zeros((1, 128), jnp.float32)
    wpack = wpack.at[0, :TOP_K].set(cnn_w)
    wpack = wpack.at[0, TOP_K].set(cnn_b[0])
    return pl.pallas_call(
        _topk_nei_kernel,
        grid=(n // _ROW_BLK,),
        in_specs=[
            pl.BlockSpec((_ROW_BLK, n), lambda b: (b, 0)),
            pl.BlockSpec((n, EMBED_DIM), lambda b: (0, 0)),
            pl.BlockSpec((1, 128), lambda b: (0, 0)),
        ],
        out_specs=pl.BlockSpec((_ROW_BLK, EMBED_DIM), lambda b: (b, 0)),
        out_shape=jax.ShapeDtypeStruct((n, EMBED_DIM), jnp.float32),
    )(sim, emb, wpack)


def _batch_kernel(ui_ref, gu_ref, gi_ref, hist_ref,
                  intW_ref, fgW_ref, Wzh_ref, Wrh_ref, Whh_ref, Wch_ref,
                  fcW_ref, wx_ref, sp_ref, out_ref):
    u_col = ui_ref[:, 0:1]
    i_col = ui_ref[:, 1:2]
    b = u_col.shape[0]
    n = gu_ref.shape[0]
    col = jax.lax.broadcasted_iota(jnp.int32, (b, n), 1)
    su = (col == u_col).astype(jnp.float32)
    uu = jnp.dot(su, gu_ref[...], preferred_element_type=jnp.float32,
                precision=jax.lax.Precision.HIGHEST)
    si = (col == i_col).astype(jnp.float32)
    vi = jnp.dot(si, gi_ref[...], preferred_element_type=jnp.float32,
                precision=jax.lax.Precision.HIGHEST)
    u_emb, u_nei = uu[:, :EMBED_DIM], uu[:, EMBED_DIM:]
    i_emb, i_nei = vi[:, :EMBED_DIM], vi[:, EMBED_DIM:]

    int_b = sp_ref[0:1, :]
    fg_b = sp_ref[1:2, :]
    b_z = sp_ref[2:3, :]
    b_r = sp_ref[3:4, :]
    b_h = sp_ref[4:5, :]
    b_c = sp_ref[5:6, :]
    fc_b = sp_ref[6:7, :]
    pred_w = sp_ref[7:8, :]
    w_zx = wx_ref[0:1, :]
    w_rx = wx_ref[1:2, :]
    w_hx = wx_ref[2:3, :]
    pred_b = wx_ref[3:4, 0:1]

    int_w = intW_ref[...]
    fg_w = fgW_ref[...]

    def interaction(a, bb):
        xv = a + bb
        fr = jax.nn.sigmoid(
            jnp.dot(xv, fg_w, preferred_element_type=jnp.float32,
                precision=jax.lax.Precision.HIGHEST) + fg_b)
        t = jnp.dot(xv, int_w, preferred_element_type=jnp.float32,
                precision=jax.lax.Precision.HIGHEST) + int_b
        return fr * t + (1.0 - fr) * xv

    cross = jnp.concatenate(
        [interaction(u_emb, u_nei), interaction(i_emb, i_nei), u_emb, i_emb],
        axis=1)
    state = jax.nn.relu(
        jnp.dot(cross, fcW_ref[...], preferred_element_type=jnp.float32,
                precision=jax.lax.Precision.HIGHEST) + fc_b)

    w_zh = Wzh_ref[...]
    w_rh = Wrh_ref[...]
    w_hh = Whh_ref[...]
    h = state
    c = jax.nn.relu(
        jnp.dot(state, Wch_ref[...], preferred_element_type=jnp.float32,
                precision=jax.lax.Precision.HIGHEST) + b_c)
    for t in range(TIME_WINDOW):
        xt = hist_ref[:, t:t + 1]
        z = jax.nn.sigmoid(
            jnp.dot(h, w_zh, preferred_element_type=jnp.float32,
                precision=jax.lax.Precision.HIGHEST)
            + xt * w_zx + b_z + c)
        rr = jax.nn.sigmoid(
            jnp.dot(h, w_rh, preferred_element_type=jnp.float32,
                precision=jax.lax.Precision.HIGHEST)
            + xt * w_rx + b_r + c)
        ht = jnp.tanh(
            jnp.dot(h * rr, w_hh, preferred_element_type=jnp.float32,
                precision=jax.lax.Precision.HIGHEST)
            + xt * w_hx + b_h + c)
        h = (1.0 - z) * h + z * ht
    y = jnp.sum(h * pred_w, axis=1, keepdims=True) + pred_b
    out_ref[...] = jnp.broadcast_to(y, (b, 128))


def kernel(x, uu_sim, ii_sim, hist, user_embedding, item_embedding,
           user_cnn_w, user_cnn_b, item_cnn_w, item_cnn_b,
           int_W, int_b, fg_W, fg_b,
           W_zh, W_zx, b_z, W_rh, W_rx, b_r, W_hh, W_hx, b_h, W_ch, b_c,
           fc_W, fc_b, pred_W, pred_b):
    batch = x.shape[0]
    n_u = uu_sim.shape[0]
    n_i = ii_sim.shape[0]
    d = EMBED_DIM

    nei_u = _neighbor_combine(uu_sim, user_embedding, user_cnn_w, user_cnn_b)
    nei_i = _neighbor_combine(ii_sim, item_embedding, item_cnn_w, item_cnn_b)

    g_u = jnp.concatenate([user_embedding, nei_u], axis=1)
    g_i = jnp.concatenate([item_embedding, nei_i], axis=1)

    ui = jnp.pad(x[:, 1:3], ((0, 0), (0, 126)))
    hist_t = hist.T

    wx = jnp.zeros((4, d), jnp.float32)
    wx = wx.at[0, :].set(W_zx[0])
    wx = wx.at[1, :].set(W_rx[0])
    wx = wx.at[2, :].set(W_hx[0])
    wx = wx.at[3, 0].set(pred_b[0])
    sp = jnp.stack([int_b, fg_b, b_z, b_r, b_h, b_c, fc_b, pred_W[:, 0]])

    full = lambda shape: pl.BlockSpec(shape, lambda bb: (0, 0))
    out = pl.pallas_call(
        _batch_kernel,
        grid=(batch // _BATCH_BLK,),
        in_specs=[
            pl.BlockSpec((_BATCH_BLK, 128), lambda bb: (bb, 0)),
            full((n_u, 2 * d)),
            full((n_i, 2 * d)),
            pl.BlockSpec((_BATCH_BLK, TIME_WINDOW), lambda bb: (bb, 0)),
            full((d, d)), full((d, d)), full((d, d)), full((d, d)),
            full((d, d)), full((d, d)), full((4 * d, d)),
            full((4, d)), full((8, d)),
        ],
        out_specs=pl.BlockSpec((_BATCH_BLK, 128), lambda bb: (bb, 0)),
        out_shape=jax.ShapeDtypeStruct((batch, 128), jnp.float32),
    )(ui, g_u, g_i, hist_t, int_W, fg_W, W_zh, W_rh, W_hh, W_ch, fc_W, wx, sp)
    return out[:, 0]
